# baseline (speedup 1.0000x reference)
import functools

import jax
import jax.numpy as jnp
from jax import lax
from jax.experimental import pallas as pl
from jax.experimental.pallas import tpu as pltpu
from jax.experimental.pallas import tpu_sc as plsc

D = 768
H = 2048
E = 8
K = 2
N = 2048
BLK = 128
NBLK = (N * K) // BLK + E
NC = 2
NS = 16
NW = NC * NS
TPW = N // NW
HTPW = TPW // 2
LANES = 16


def _router_body(x_ref, rw_ref, rb_ref, dest_ref, wn_ref, meta_ref):
    xf = x_ref[...]
    rw = rw_ref[...]
    logits = lax.dot_general(xf, rw, (((1,), (1,)), ((), ())),
                             preferred_element_type=jnp.float32)
    logits = logits + rb_ref[...]
    m = jnp.max(logits, axis=1, keepdims=True)
    p = jnp.exp(logits - m)
    probs = p / jnp.sum(p, axis=1, keepdims=True)

    lane = lax.broadcasted_iota(jnp.int32, (N, E), 1)
    idx0 = jnp.argmax(probs, axis=1).astype(jnp.int32)
    v0 = jnp.max(probs, axis=1)
    oh0 = lane == idx0[:, None]
    probs2 = jnp.where(oh0, -1.0, probs)
    idx1 = jnp.argmax(probs2, axis=1).astype(jnp.int32)
    v1 = jnp.max(probs2, axis=1)
    oh1 = lane == idx1[:, None]

    wsum = v0 + v1 + 1e-9
    w0 = v0 / wsum
    w1 = v1 / wsum
    wn_ref[...] = jnp.concatenate([w0[None, :], w1[None, :]], axis=0)

    occ = oh0.astype(jnp.float32) + oh1.astype(jnp.float32)
    ri = lax.broadcasted_iota(jnp.int32, (N, N), 0)
    ci = lax.broadcasted_iota(jnp.int32, (N, N), 1)
    ltri = (ci < ri).astype(jnp.float32)
    rank_full = lax.dot_general(ltri, occ, (((1,), (0,)), ((), ())),
                                preferred_element_type=jnp.float32)

    counts = jnp.sum(occ, axis=0, keepdims=True)
    counts_i = counts.astype(jnp.int32)
    bpe = (counts_i + (BLK - 1)) // BLK
    ui = lax.broadcasted_iota(jnp.int32, (E, E), 0)
    uj = lax.broadcasted_iota(jnp.int32, (E, E), 1)
    utri = (ui <= uj).astype(jnp.float32)
    cum_bpe = lax.dot_general(bpe.astype(jnp.float32), utri,
                              (((1,), (0,)), ((), ())),
                              preferred_element_type=jnp.float32)
    off = (cum_bpe - bpe.astype(jnp.float32)) * BLK

    rank0 = jnp.sum(jnp.where(oh0, rank_full, 0.0), axis=1)
    rank1 = jnp.sum(jnp.where(oh1, rank_full, 0.0), axis=1)
    off0 = jnp.sum(jnp.where(oh0, off, 0.0), axis=1)
    off1 = jnp.sum(jnp.where(oh1, off, 0.0), axis=1)
    dest0 = (off0 + rank0).astype(jnp.int32)
    dest1 = (off1 + rank1).astype(jnp.int32)
    dest_ref[...] = jnp.concatenate([dest0[None, :], dest1[None, :]], axis=0)

    cum_i = cum_bpe.astype(jnp.int32)
    bb = lax.broadcasted_iota(jnp.int32, (NBLK, E), 0)
    be_raw = jnp.sum((bb >= cum_i).astype(jnp.int32), axis=1)
    lane_e = lax.broadcasted_iota(jnp.int32, (1, E), 1)
    last_e = jnp.max(jnp.where(counts_i > 0, lane_e, 0))
    be = jnp.minimum(be_raw, last_e)
    total = cum_i[0, E - 1]
    pad = jnp.zeros((64 - NBLK,), jnp.int32)
    meta = jnp.concatenate([be, pad])[None, :]
    pos = lax.broadcasted_iota(jnp.int32, (1, 64), 1)
    meta_ref[...] = jnp.where(pos == NBLK, total, meta)


def _router(flat, router_w, router_b):
    return pl.pallas_call(
        _router_body,
        out_shape=[
            jax.ShapeDtypeStruct((K, N), jnp.int32),
            jax.ShapeDtypeStruct((K, N), jnp.float32),
            jax.ShapeDtypeStruct((1, 64), jnp.int32),
        ],
    )(flat, router_w, router_b.reshape(1, E))


@functools.lru_cache(maxsize=None)
def _make_scatter_x():
    @functools.partial(
        pl.kernel,
        mesh=plsc.VectorSubcoreMesh(
            core_axis_name="c", subcore_axis_name="s", num_cores=NC),
        out_type=jax.ShapeDtypeStruct((NBLK * BLK, D), jnp.float32),
        scratch_types=[
            pltpu.VMEM((TPW, D), jnp.float32),
            pltpu.VMEM((TPW,), jnp.int32),
            pltpu.VMEM((TPW,), jnp.int32),
            pltpu.SemaphoreType.DMA,
            pltpu.SemaphoreType.DMA,
        ],
    )
    def _scatter_x(x_hbm, dest_hbm, xs_hbm, rows_v, i0_v, i1_v, sem0, sem1):
        wid = lax.axis_index("s") * NC + lax.axis_index("c")
        t0 = wid * TPW
        pltpu.sync_copy(dest_hbm.at[0, pl.ds(t0, TPW)], i0_v)
        pltpu.sync_copy(dest_hbm.at[1, pl.ds(t0, TPW)], i1_v)
        pltpu.sync_copy(x_hbm.at[pl.ds(t0, TPW)], rows_v)
        c0 = pltpu.async_copy(rows_v, xs_hbm.at[i0_v], sem0)
        c1 = pltpu.async_copy(rows_v, xs_hbm.at[i1_v], sem1)
        c0.wait()
        c1.wait()

    return _scatter_x


def _moe_body(be_ref, xs_ref, gw_ref, uw_ref, dw_ref, y_ref):
    b = pl.program_id(0)
    total = be_ref[NBLK]

    @pl.when(b < total)
    def _():
        xb = xs_ref[...].astype(jnp.bfloat16)
        g = lax.dot_general(xb, gw_ref[0], (((1,), (1,)), ((), ())),
                            preferred_element_type=jnp.float32)
        u = lax.dot_general(xb, uw_ref[0], (((1,), (1,)), ((), ())),
                            preferred_element_type=jnp.float32)
        h = (g * jax.nn.sigmoid(g) * u).astype(jnp.bfloat16)
        y_ref[...] = lax.dot_general(h, dw_ref[0], (((1,), (1,)), ((), ())),
                                     preferred_element_type=jnp.float32)


def _moe_mm(be_vec, xs, gate_w, up_w, down_w):
    grid_spec = pltpu.PrefetchScalarGridSpec(
        num_scalar_prefetch=1,
        grid=(NBLK,),
        in_specs=[
            pl.BlockSpec((BLK, D), lambda b, be: (b, 0)),
            pl.BlockSpec((1, H, D), lambda b, be: (be[b], 0, 0)),
            pl.BlockSpec((1, H, D), lambda b, be: (be[b], 0, 0)),
            pl.BlockSpec((1, D, H), lambda b, be: (be[b], 0, 0)),
        ],
        out_specs=pl.BlockSpec((BLK, D), lambda b, be: (b, 0)),
    )
    return pl.pallas_call(
        _moe_body,
        grid_spec=grid_spec,
        out_shape=jax.ShapeDtypeStruct((NBLK * BLK, D), jnp.float32),
    )(be_vec, xs, gate_w, up_w, down_w)


@functools.lru_cache(maxsize=None)
def _make_combine():
    @functools.partial(
        pl.kernel,
        mesh=plsc.VectorSubcoreMesh(
            core_axis_name="c", subcore_axis_name="s", num_cores=NC),
        out_type=jax.ShapeDtypeStruct((N, D), jnp.float32),
        scratch_types=[
            pltpu.VMEM((HTPW, D), jnp.float32),
            pltpu.VMEM((HTPW, D), jnp.float32),
            pltpu.VMEM((HTPW,), jnp.int32),
            pltpu.VMEM((HTPW,), jnp.int32),
            pltpu.VMEM((HTPW, LANES), jnp.float32),
            pltpu.VMEM((HTPW, LANES), jnp.float32),
            pltpu.SemaphoreType.DMA,
            pltpu.SemaphoreType.DMA,
        ],
    )
    def _combine(y_hbm, dest_hbm, wn_hbm, out_hbm,
                 r0_v, r1_v, i0_v, i1_v, w0_v, w1_v, sem0, sem1):
        wid = lax.axis_index("s") * NC + lax.axis_index("c")
        for half in range(2):
            t0 = wid * TPW + half * HTPW
            pltpu.sync_copy(dest_hbm.at[0, pl.ds(t0, HTPW)], i0_v)
            pltpu.sync_copy(dest_hbm.at[1, pl.ds(t0, HTPW)], i1_v)
            pltpu.sync_copy(wn_hbm.at[0, pl.ds(t0, HTPW)], w0_v)
            pltpu.sync_copy(wn_hbm.at[1, pl.ds(t0, HTPW)], w1_v)
            a0 = pltpu.async_copy(y_hbm.at[i0_v], r0_v, sem0)
            a1 = pltpu.async_copy(y_hbm.at[i1_v], r1_v, sem1)
            a0.wait()
            a1.wait()

            def row_body(r, _):
                w0 = w0_v[r, :]
                w1 = w1_v[r, :]
                for c in range(D // LANES):
                    sl = pl.ds(c * LANES, LANES)
                    r0_v[r, sl] = w0 * r0_v[r, sl] + w1 * r1_v[r, sl]
                return 0

            lax.fori_loop(0, HTPW, row_body, 0)
            pltpu.sync_copy(r0_v, out_hbm.at[pl.ds(t0, HTPW)])

    return _combine


def kernel(x, router_w, router_b, gate_w, up_w, down_w):
    Bb, Tt, Dd = x.shape
    flat = x.reshape(N, D)
    dest, wn, meta = _router(flat, router_w, router_b)
    be_vec = meta.reshape(-1)[: NBLK + 1]
    xs = _make_scatter_x()(flat, dest)
    y = _moe_mm(be_vec, xs, gate_w.astype(jnp.bfloat16),
                up_w.astype(jnp.bfloat16), down_w.astype(jnp.bfloat16))
    wn_pad = jnp.broadcast_to(wn[:, :, None], (K, N, LANES))
    out = _make_combine()(y, dest, wn_pad)
    return out.reshape(Bb, Tt, Dd)
